# baseline (speedup 1.0000x reference)
import functools

import jax
import jax.numpy as jnp
from jax import lax
from jax.experimental import pallas as pl
from jax.experimental.pallas import tpu as pltpu
from jax.experimental.pallas import tpu_sc as plsc

N = 10000
E = 320000
H = 128
HH = H // 2
G = 32
N_LAYERS = 3

NC = 2
NS = 16
NW = NC * NS
EPW = E // NW
C = 80
NCH = EPW // C
EPAIR = E // 2
PPT = EPAIR // NW
CP = C // 2
N_PAD = 10240
RPT = N_PAD // NS
RCH = 128


def _sc_message_pass(h0, h1, ep0, ep1, src_r, dst_r):
  mesh = plsc.VectorSubcoreMesh(core_axis_name="c", subcore_axis_name="s")
  half_out = jax.ShapeDtypeStruct((NC, N_PAD, HH), jnp.float32)

  @functools.partial(
      pl.kernel,
      out_type=(half_out, half_out),
      mesh=mesh,
      compiler_params=pltpu.CompilerParams(use_tc_tiling_on_sc=False),
      scratch_types=[
          pltpu.VMEM((NCH, C), jnp.int32),
          pltpu.VMEM((NCH, C), jnp.int32),
          pltpu.VMEM((CP, H), jnp.float32),
          pltpu.VMEM((CP, H), jnp.float32),
          pltpu.VMEM((C, HH), jnp.float32),
          pltpu.VMEM((C, HH), jnp.float32),
          pltpu.VMEM((C, HH), jnp.float32),
          pltpu.VMEM((RCH, HH), jnp.float32),
          pltpu.VMEM_SHARED((N_PAD, HH), jnp.float32),
          pltpu.SemaphoreType.DMA,
          pltpu.SemaphoreType.DMA,
          pltpu.SemaphoreType.DMA,
          pltpu.SemaphoreType.DMA,
      ],
  )
  def k(h0_hbm, h1_hbm, ep0_hbm, ep1_hbm, src_hbm, dst_hbm, out0_hbm, out1_hbm,
        src_v, dst_v, p0, p1, g0, g1, msg_v, z_v, acc, se0, se1, sg0, sg1):
    c = lax.axis_index("c")
    s = lax.axis_index("s")
    wid = c * NS + s
    pbase = wid * PPT

    pltpu.sync_copy(src_hbm.at[wid], src_v)
    pltpu.sync_copy(dst_hbm.at[wid], dst_v)

    zero = jnp.zeros((16,), jnp.float32)

    @pl.loop(0, RCH)
    def _(r):
      for q in range(HH // 16):
        z_v[r, pl.ds(q * 16, 16)] = zero

    pb = (p0, p1)
    gb = (g0, g1)
    se = (se0, se1)
    sg = (sg0, sg1)

    for h_hbm, ep_hbm, out_hbm in ((h0_hbm, ep0_hbm, out0_hbm),
                                   (h1_hbm, ep1_hbm, out1_hbm)):
      for t in range(RPT // RCH):
        pltpu.sync_copy(z_v, acc.at[pl.ds(s * RPT + t * RCH, RCH)])
      plsc.subcore_barrier()

      for b in range(2):
        pltpu.async_copy(ep_hbm.at[pl.ds(pbase + b * CP, CP)], pb[b], se[b])
        pltpu.async_copy(h_hbm.at[src_v.at[b]], gb[b], sg[b])

      def process(jj, b, prefetch):
        pltpu.make_async_copy(
            ep_hbm.at[pl.ds(pbase + jj * CP, CP)], pb[b], se[b]).wait()
        pltpu.make_async_copy(h_hbm.at[src_v.at[jj]], gb[b], sg[b]).wait()

        @plsc.parallel_loop(0, CP, step=1, unroll=4)
        def _(r):
          for sub in range(2):
            e = 2 * r + sub
            for q in range(HH // 16):
              msg_v[e, pl.ds(q * 16, 16)] = jnp.maximum(
                  pb[b][r, pl.ds(sub * HH + q * 16, 16)]
                  + gb[b][e, pl.ds(q * 16, 16)],
                  0.0,
              )

        pltpu.sync_copy(msg_v, acc.at[dst_v.at[jj]], add=True)
        if prefetch:

          @pl.when(jj + 2 < NCH)
          def _():
            pltpu.async_copy(
                ep_hbm.at[pl.ds(pbase + (jj + 2) * CP, CP)], pb[b], se[b])
            pltpu.async_copy(h_hbm.at[src_v.at[jj + 2]], gb[b], sg[b])

      @pl.loop(0, NCH - 1, step=2)
      def _(j):
        for b in range(2):
          process(j + b, b, prefetch=True)

      process(NCH - 1, (NCH - 1) % 2, prefetch=False)
      plsc.subcore_barrier()

      for t in range(RPT // RCH):
        sl = pl.ds(s * RPT + t * RCH, RCH)
        pltpu.sync_copy(acc.at[sl], out_hbm.at[c, sl])

  return k(h0, h1, ep0, ep1, src_r, dst_r)


def _in_proj_body(x_ref, w_ref, b_ref, of_ref, o0_ref, o1_ref):
  t = (
      jnp.dot(x_ref[...], w_ref[...], preferred_element_type=jnp.float32)
      + b_ref[...]
  )
  of_ref[...] = t
  o0_ref[...] = t[:, :HH]
  o1_ref[...] = t[:, HH:]


def _in_proj(x, w, b):
  bm = 2000
  return pl.pallas_call(
      _in_proj_body,
      grid=(N // bm,),
      in_specs=[
          pl.BlockSpec((bm, H), lambda i: (i, 0)),
          pl.BlockSpec((H, H), lambda i: (0, 0)),
          pl.BlockSpec((1, H), lambda i: (0, 0)),
      ],
      out_specs=[
          pl.BlockSpec((bm, H), lambda i: (i, 0)),
          pl.BlockSpec((bm, HH), lambda i: (i, 0)),
          pl.BlockSpec((bm, HH), lambda i: (i, 0)),
      ],
      out_shape=[
          jax.ShapeDtypeStruct((N, H), jnp.float32),
          jax.ShapeDtypeStruct((N, HH), jnp.float32),
          jax.ShapeDtypeStruct((N, HH), jnp.float32),
      ],
  )(x, w, b)


def _edge_proj_body(ea_ref, w0_ref, w1_ref, o0_ref, o1_ref):
  ea = ea_ref[...].astype(jnp.bfloat16)
  dn = (((0,), (0,)), ((), ()))
  o0_ref[...] = lax.dot_general(ea, w0_ref[...], dn,
                                preferred_element_type=jnp.float32)
  o1_ref[...] = lax.dot_general(ea, w1_ref[...], dn,
                                preferred_element_type=jnp.float32)


def _edge_proj(ea_pair_t, w):
  zz = jnp.zeros((16, HH), jnp.bfloat16)
  wb = w.astype(jnp.bfloat16)

  def pairw(wh):
    a = jnp.concatenate([wh, zz], axis=1)
    b = jnp.concatenate([zz, wh], axis=1)
    return jnp.stack([a, b], axis=1).reshape(32, H)

  w0p = pairw(wb[:, :HH])
  w1p = pairw(wb[:, HH:])
  bm = 6400
  return pl.pallas_call(
      _edge_proj_body,
      grid=(EPAIR // bm,),
      in_specs=[
          pl.BlockSpec((32, bm), lambda i: (0, i)),
          pl.BlockSpec((32, H), lambda i: (0, 0)),
          pl.BlockSpec((32, H), lambda i: (0, 0)),
      ],
      out_specs=[
          pl.BlockSpec((bm, H), lambda i: (i, 0)),
          pl.BlockSpec((bm, H), lambda i: (i, 0)),
      ],
      out_shape=[
          jax.ShapeDtypeStruct((EPAIR, H), jnp.float32),
          jax.ShapeDtypeStruct((EPAIR, H), jnp.float32),
      ],
  )(ea_pair_t, w0p, w1p)


def _update_body(h0_ref, h1_ref, a00_ref, a01_ref, a10_ref, a11_ref,
                 w0_ref, w1_ref, b_ref, of_ref, o0_ref, o1_ref):
  t0 = h0_ref[...] + a00_ref[...] + a01_ref[...]
  t1 = h1_ref[...] + a10_ref[...] + a11_ref[...]
  t = jnp.maximum(
      jnp.dot(t0, w0_ref[...], preferred_element_type=jnp.float32)
      + jnp.dot(t1, w1_ref[...], preferred_element_type=jnp.float32)
      + b_ref[...],
      0.0,
  )
  of_ref[...] = t
  o0_ref[...] = t[:, :HH]
  o1_ref[...] = t[:, HH:]


def _update(h0, h1, a0, a1, w, b):
  bm = 2000
  half_spec = pl.BlockSpec((bm, HH), lambda i: (i, 0))
  return pl.pallas_call(
      _update_body,
      grid=(N // bm,),
      in_specs=[
          half_spec,
          half_spec,
          half_spec,
          half_spec,
          half_spec,
          half_spec,
          pl.BlockSpec((HH, H), lambda i: (0, 0)),
          pl.BlockSpec((HH, H), lambda i: (0, 0)),
          pl.BlockSpec((1, H), lambda i: (0, 0)),
      ],
      out_specs=[
          pl.BlockSpec((bm, H), lambda i: (i, 0)),
          half_spec,
          half_spec,
      ],
      out_shape=[
          jax.ShapeDtypeStruct((N, H), jnp.float32),
          jax.ShapeDtypeStruct((N, HH), jnp.float32),
          jax.ShapeDtypeStruct((N, HH), jnp.float32),
      ],
  )(h0, h1, a0[0], a0[1], a1[0], a1[1], w[:HH], w[HH:], b)


def _layer_norm(v, g, b):
  mu = jnp.mean(v, axis=1, keepdims=True)
  var = jnp.mean((v - mu) ** 2, axis=1, keepdims=True)
  return (v - mu) / jnp.sqrt(var + 1e-5) * g + b


def _pool_head_body(h_ref, seg_ref, gw_ref, gb_ref, ag_ref, ab_ref, aw_ref,
                    abias_ref, l1g_ref, l1b_ref, w1_ref, b1_ref, l2g_ref,
                    l2b_ref, w2_ref, b2_ref, o_ref):
  h = h_ref[...]
  gate = jnp.sum(h * gw_ref[...], axis=1, keepdims=True) + gb_ref[0, 0]
  feat = _layer_norm(h, ag_ref[...], ab_ref[...])
  feat = (
      jnp.dot(jnp.maximum(feat, 0.0), aw_ref[...],
              preferred_element_type=jnp.float32)
      + abias_ref[...]
  )
  seg = seg_ref[...]
  gids = lax.broadcasted_iota(jnp.int32, (1, G), 1)
  onehot = seg == gids
  onef = onehot.astype(jnp.float32)
  m = jnp.max(jnp.where(onehot, gate, jnp.float32(-1e30)), axis=0,
              keepdims=True)
  mb = jnp.sum(onef * m, axis=1, keepdims=True)
  eg = jnp.exp(gate - mb)
  denom = jnp.sum(onef * eg, axis=0, keepdims=True)
  db = jnp.sum(onef * denom, axis=1, keepdims=True)
  alpha = eg / (db + 1e-16)
  pooled = lax.dot_general(
      onef, alpha * feat, (((0,), (0,)), ((), ())),
      preferred_element_type=jnp.float32)
  o = _layer_norm(pooled, l1g_ref[...], l1b_ref[...])
  o = (
      jnp.dot(jnp.maximum(o, 0.0), w1_ref[...],
              preferred_element_type=jnp.float32)
      + b1_ref[...]
  )
  o = _layer_norm(o, l2g_ref[...], l2b_ref[...])
  o = (
      jnp.dot(jnp.maximum(o, 0.0), w2_ref[...],
              preferred_element_type=jnp.float32)
      + b2_ref[...]
  )
  o_ref[...] = o


def _pool_head(h, seg, p):
  args = (
      h,
      seg,
      p["gate_W"].reshape(1, H),
      p["gate_b"].reshape(1, 1),
      p["aff_g"].reshape(1, H),
      p["aff_b"].reshape(1, H),
      p["aff_W"],
      p["aff_bias"].reshape(1, H),
      p["ln1_g"].reshape(1, H),
      p["ln1_b"].reshape(1, H),
      p["W1"],
      p["b1"].reshape(1, H // 2),
      p["ln2_g"].reshape(1, H // 2),
      p["ln2_b"].reshape(1, H // 2),
      p["W2"],
      p["b2"].reshape(1, 4),
  )
  return pl.pallas_call(
      _pool_head_body,
      out_shape=jax.ShapeDtypeStruct((G, 4), jnp.float32),
  )(*args)


def kernel(x, edge_index, edge_attr, batch, params):
  ei_p = jnp.stack(
      [edge_index[:, :EPAIR], edge_index[:, EPAIR:]], axis=2)
  src_r = ei_p[0].reshape(NW, NCH, C)
  dst_r = ei_p[1].reshape(NW, NCH, C)
  ea_pair_t = jnp.stack(
      [edge_attr[:EPAIR], edge_attr[EPAIR:]], axis=2).reshape(EPAIR, 32).T
  h, h0, h1 = _in_proj(x, params["W_in"], params["b_in"].reshape(1, H))
  for l in range(N_LAYERS):
    ep0, ep1 = _edge_proj(ea_pair_t, params["We"][l])
    a0, a1 = _sc_message_pass(h0, h1, ep0, ep1, src_r, dst_r)
    h, h0, h1 = _update(h0, h1, a0, a1, params["W"][l],
                        params["b"][l].reshape(1, H))
  return _pool_head(h, batch.reshape(N, 1).astype(jnp.int32), params)


# speedup vs baseline: 1.0572x; 1.0572x over previous
import functools

import jax
import jax.numpy as jnp
from jax import lax
from jax.experimental import pallas as pl
from jax.experimental.pallas import tpu as pltpu
from jax.experimental.pallas import tpu_sc as plsc

N = 10000
E = 320000
H = 128
HH = H // 2
G = 32
N_LAYERS = 3

NC = 2
NS = 16
NW = NC * NS
EPW = E // NW
C = 80
NCH = EPW // C
EPAIR = E // 2
PPT = EPAIR // NW
CP = C // 2
N_PAD = 10240
RPT = N_PAD // NS
RCH = 128


def _sc_message_pass(h0, h1, ep0, ep1, src_r, dst_r):
  mesh = plsc.VectorSubcoreMesh(core_axis_name="c", subcore_axis_name="s")
  half_out = jax.ShapeDtypeStruct((NC, N_PAD, HH), jnp.float32)

  @functools.partial(
      pl.kernel,
      out_type=(half_out, half_out),
      mesh=mesh,
      compiler_params=pltpu.CompilerParams(use_tc_tiling_on_sc=False),
      scratch_types=[
          pltpu.VMEM((NCH, C), jnp.int32),
          pltpu.VMEM((NCH, C), jnp.int32),
          pltpu.VMEM((CP, H), jnp.float32),
          pltpu.VMEM((CP, H), jnp.float32),
          pltpu.VMEM((C, HH), jnp.float32),
          pltpu.VMEM((C, HH), jnp.float32),
          pltpu.VMEM((C, HH), jnp.float32),
          pltpu.VMEM((RCH, HH), jnp.float32),
          pltpu.VMEM_SHARED((N_PAD, HH), jnp.float32),
          pltpu.SemaphoreType.DMA,
          pltpu.SemaphoreType.DMA,
          pltpu.SemaphoreType.DMA,
          pltpu.SemaphoreType.DMA,
      ],
  )
  def k(h0_hbm, h1_hbm, ep0_hbm, ep1_hbm, src_hbm, dst_hbm, out0_hbm, out1_hbm,
        src_v, dst_v, p0, p1, g0, g1, msg_v, z_v, acc, se0, se1, sg0, sg1):
    c = lax.axis_index("c")
    s = lax.axis_index("s")
    wid = c * NS + s
    pbase = wid * PPT

    pltpu.sync_copy(src_hbm.at[wid], src_v)
    pltpu.sync_copy(dst_hbm.at[wid], dst_v)

    zero = jnp.zeros((16,), jnp.float32)

    @pl.loop(0, RCH)
    def _(r):
      for q in range(HH // 16):
        z_v[r, pl.ds(q * 16, 16)] = zero

    pb = (p0, p1)
    gb = (g0, g1)
    se = (se0, se1)
    sg = (sg0, sg1)

    for h_hbm, ep_hbm, out_hbm in ((h0_hbm, ep0_hbm, out0_hbm),
                                   (h1_hbm, ep1_hbm, out1_hbm)):
      for t in range(RPT // RCH):
        pltpu.sync_copy(z_v, acc.at[pl.ds(s * RPT + t * RCH, RCH)])
      plsc.subcore_barrier()

      for b in range(2):
        pltpu.async_copy(ep_hbm.at[pl.ds(pbase + b * CP, CP)], pb[b], se[b])
        pltpu.async_copy(h_hbm.at[src_v.at[b]], gb[b], sg[b])

      def process(jj, b, prefetch):
        pltpu.make_async_copy(
            ep_hbm.at[pl.ds(pbase + jj * CP, CP)], pb[b], se[b]).wait()
        pltpu.make_async_copy(h_hbm.at[src_v.at[jj]], gb[b], sg[b]).wait()

        @plsc.parallel_loop(0, CP, step=1, unroll=4)
        def _(r):
          for sub in range(2):
            e = 2 * r + sub
            for q in range(HH // 16):
              msg_v[e, pl.ds(q * 16, 16)] = jnp.maximum(
                  pb[b][r, pl.ds(sub * HH + q * 16, 16)]
                  + gb[b][e, pl.ds(q * 16, 16)],
                  0.0,
              )

        pltpu.sync_copy(msg_v, acc.at[dst_v.at[jj]], add=True)
        if prefetch:

          @pl.when(jj + 2 < NCH)
          def _():
            pltpu.async_copy(
                ep_hbm.at[pl.ds(pbase + (jj + 2) * CP, CP)], pb[b], se[b])
            pltpu.async_copy(h_hbm.at[src_v.at[jj + 2]], gb[b], sg[b])

      @pl.loop(0, NCH - 1, step=2)
      def _(j):
        for b in range(2):
          process(j + b, b, prefetch=True)

      process(NCH - 1, (NCH - 1) % 2, prefetch=False)
      plsc.subcore_barrier()

      for t in range(RPT // RCH):
        sl = pl.ds(s * RPT + t * RCH, RCH)
        pltpu.sync_copy(acc.at[sl], out_hbm.at[c, sl])

  return k(h0, h1, ep0, ep1, src_r, dst_r)


def _in_proj_body(x_ref, w_ref, b_ref, of_ref, o0_ref, o1_ref):
  t = (
      jnp.dot(x_ref[...], w_ref[...], preferred_element_type=jnp.float32)
      + b_ref[...]
  )
  of_ref[...] = t
  o0_ref[...] = t[:, :HH]
  o1_ref[...] = t[:, HH:]


def _in_proj(x, w, b):
  bm = 2000
  return pl.pallas_call(
      _in_proj_body,
      grid=(N // bm,),
      in_specs=[
          pl.BlockSpec((bm, H), lambda i: (i, 0)),
          pl.BlockSpec((H, H), lambda i: (0, 0)),
          pl.BlockSpec((1, H), lambda i: (0, 0)),
      ],
      out_specs=[
          pl.BlockSpec((bm, H), lambda i: (i, 0)),
          pl.BlockSpec((bm, HH), lambda i: (i, 0)),
          pl.BlockSpec((bm, HH), lambda i: (i, 0)),
      ],
      out_shape=[
          jax.ShapeDtypeStruct((N, H), jnp.float32),
          jax.ShapeDtypeStruct((N, HH), jnp.float32),
          jax.ShapeDtypeStruct((N, HH), jnp.float32),
      ],
  )(x, w, b)


def _edge_proj_body(ea_ref, w0_ref, w1_ref, o0_ref, o1_ref):
  ea = ea_ref[...].astype(jnp.bfloat16)
  o0_ref[...] = jnp.dot(ea, w0_ref[...], preferred_element_type=jnp.float32)
  o1_ref[...] = jnp.dot(ea, w1_ref[...], preferred_element_type=jnp.float32)


def _edge_proj(ea_pair, w):
  zz = jnp.zeros((16, HH), jnp.bfloat16)
  wb = w.astype(jnp.bfloat16)
  w0p = jnp.concatenate(
      [jnp.concatenate([wb[:, :HH], zz], axis=1),
       jnp.concatenate([zz, wb[:, :HH]], axis=1)], axis=0)
  w1p = jnp.concatenate(
      [jnp.concatenate([wb[:, HH:], zz], axis=1),
       jnp.concatenate([zz, wb[:, HH:]], axis=1)], axis=0)
  bm = 4000
  return pl.pallas_call(
      _edge_proj_body,
      grid=(EPAIR // bm,),
      in_specs=[
          pl.BlockSpec((bm, 32), lambda i: (i, 0)),
          pl.BlockSpec((32, H), lambda i: (0, 0)),
          pl.BlockSpec((32, H), lambda i: (0, 0)),
      ],
      out_specs=[
          pl.BlockSpec((bm, H), lambda i: (i, 0)),
          pl.BlockSpec((bm, H), lambda i: (i, 0)),
      ],
      out_shape=[
          jax.ShapeDtypeStruct((EPAIR, H), jnp.float32),
          jax.ShapeDtypeStruct((EPAIR, H), jnp.float32),
      ],
  )(ea_pair, w0p, w1p)


def _update_body(h0_ref, h1_ref, a0_ref, a1_ref,
                 w0_ref, w1_ref, b_ref, of_ref, o0_ref, o1_ref):
  t0 = h0_ref[...] + a0_ref[0] + a0_ref[1]
  t1 = h1_ref[...] + a1_ref[0] + a1_ref[1]
  t = jnp.maximum(
      jnp.dot(t0, w0_ref[...], preferred_element_type=jnp.float32)
      + jnp.dot(t1, w1_ref[...], preferred_element_type=jnp.float32)
      + b_ref[...],
      0.0,
  )
  of_ref[...] = t
  o0_ref[...] = t[:, :HH]
  o1_ref[...] = t[:, HH:]


def _update(h0, h1, a0, a1, w, b):
  bm = 2000
  half_spec = pl.BlockSpec((bm, HH), lambda i: (i, 0))
  acc_spec = pl.BlockSpec((NC, bm, HH), lambda i: (0, i, 0))
  return pl.pallas_call(
      _update_body,
      grid=(N // bm,),
      in_specs=[
          half_spec,
          half_spec,
          acc_spec,
          acc_spec,
          pl.BlockSpec((HH, H), lambda i: (0, 0)),
          pl.BlockSpec((HH, H), lambda i: (0, 0)),
          pl.BlockSpec((1, H), lambda i: (0, 0)),
      ],
      out_specs=[
          pl.BlockSpec((bm, H), lambda i: (i, 0)),
          half_spec,
          half_spec,
      ],
      out_shape=[
          jax.ShapeDtypeStruct((N, H), jnp.float32),
          jax.ShapeDtypeStruct((N, HH), jnp.float32),
          jax.ShapeDtypeStruct((N, HH), jnp.float32),
      ],
  )(h0, h1, a0, a1, w[:HH], w[HH:], b)


def _layer_norm(v, g, b):
  mu = jnp.mean(v, axis=1, keepdims=True)
  var = jnp.mean((v - mu) ** 2, axis=1, keepdims=True)
  return (v - mu) / jnp.sqrt(var + 1e-5) * g + b


def _pool_head_body(h_ref, seg_ref, gw_ref, gb_ref, ag_ref, ab_ref, aw_ref,
                    abias_ref, l1g_ref, l1b_ref, w1_ref, b1_ref, l2g_ref,
                    l2b_ref, w2_ref, b2_ref, o_ref):
  h = h_ref[...]
  gate = jnp.sum(h * gw_ref[...], axis=1, keepdims=True) + gb_ref[0, 0]
  feat = _layer_norm(h, ag_ref[...], ab_ref[...])
  feat = (
      jnp.dot(jnp.maximum(feat, 0.0), aw_ref[...],
              preferred_element_type=jnp.float32)
      + abias_ref[...]
  )
  seg = seg_ref[...]
  gids = lax.broadcasted_iota(jnp.int32, (1, G), 1)
  onehot = seg == gids
  onef = onehot.astype(jnp.float32)
  m = jnp.max(jnp.where(onehot, gate, jnp.float32(-1e30)), axis=0,
              keepdims=True)
  mb = jnp.sum(onef * m, axis=1, keepdims=True)
  eg = jnp.exp(gate - mb)
  denom = jnp.sum(onef * eg, axis=0, keepdims=True)
  db = jnp.sum(onef * denom, axis=1, keepdims=True)
  alpha = eg / (db + 1e-16)
  pooled = lax.dot_general(
      onef, alpha * feat, (((0,), (0,)), ((), ())),
      preferred_element_type=jnp.float32)
  o = _layer_norm(pooled, l1g_ref[...], l1b_ref[...])
  o = (
      jnp.dot(jnp.maximum(o, 0.0), w1_ref[...],
              preferred_element_type=jnp.float32)
      + b1_ref[...]
  )
  o = _layer_norm(o, l2g_ref[...], l2b_ref[...])
  o = (
      jnp.dot(jnp.maximum(o, 0.0), w2_ref[...],
              preferred_element_type=jnp.float32)
      + b2_ref[...]
  )
  o_ref[...] = o


def _pool_head(h, seg, p):
  args = (
      h,
      seg,
      p["gate_W"].reshape(1, H),
      p["gate_b"].reshape(1, 1),
      p["aff_g"].reshape(1, H),
      p["aff_b"].reshape(1, H),
      p["aff_W"],
      p["aff_bias"].reshape(1, H),
      p["ln1_g"].reshape(1, H),
      p["ln1_b"].reshape(1, H),
      p["W1"],
      p["b1"].reshape(1, H // 2),
      p["ln2_g"].reshape(1, H // 2),
      p["ln2_b"].reshape(1, H // 2),
      p["W2"],
      p["b2"].reshape(1, 4),
  )
  return pl.pallas_call(
      _pool_head_body,
      out_shape=jax.ShapeDtypeStruct((G, 4), jnp.float32),
  )(*args)


def kernel(x, edge_index, edge_attr, batch, params):
  src_r = edge_index[0].reshape(NW, NCH, C)
  dst_r = edge_index[1].reshape(NW, NCH, C)
  ea_pair = edge_attr.reshape(EPAIR, 32)
  h, h0, h1 = _in_proj(x, params["W_in"], params["b_in"].reshape(1, H))
  for l in range(N_LAYERS):
    ep0, ep1 = _edge_proj(ea_pair, params["We"][l])
    a0, a1 = _sc_message_pass(h0, h1, ep0, ep1, src_r, dst_r)
    h, h0, h1 = _update(h0, h1, a0, a1, params["W"][l],
                        params["b"][l].reshape(1, H))
  return _pool_head(h, batch.reshape(N, 1).astype(jnp.int32), params)
